# baseline (speedup 1.0000x reference)
import functools

import jax
import jax.numpy as jnp
from jax import lax
from jax.experimental import pallas as pl
from jax.experimental.pallas import tpu as pltpu
from jax.experimental.pallas import tpu_sc as plsc

N = 100000
E = 3200000
NP = 100096
STRIPE = 6256
EPW = E // 32
CHUNK = 800
NCHUNK = EPW // CHUNK
GB = 80
EPS = 1e-5


_NR = N * 16 // 128
_NRP = NP * 16 // 128


def _k1_body(p_ref, v_ref, wp_ref, wv_ref, g_ref):
    g_ref[...] = (
        jnp.dot(p_ref[...], wp_ref[...], preferred_element_type=jnp.float32)
        + jnp.dot(v_ref[...], wv_ref[...], preferred_element_type=jnp.float32))


def _k1(pos64, vel64, wp, wv):
    return pl.pallas_call(
        _k1_body,
        out_shape=jax.ShapeDtypeStruct((_NR, 128), jnp.float32),
    )(pos64, vel64, wp, wv)


def _k2_body(g_hbm, edge2, zcnt, y1_hbm, part_hbm, cnt_out,
             sbi0, dbi0, sbi1, dbi1, sr0, dr0, sr1, dr1, difb0, difb1,
             accv, onesb, cnt_sh,
             semI0, semI1, semG0, semG1, semW0, semW1, semC0, semC1):
    c_i = lax.axis_index("c")
    s_i = lax.axis_index("s")
    wid = s_i * 2 + c_i
    r0 = s_i * STRIPE
    RPC = CHUNK // GB

    sbi = (sbi0, sbi1)
    dbi = (dbi0, dbi1)
    sr = (sr0, sr1)
    dr = (dr0, dr1)
    difb = (difb0, difb1)
    semI = (semI0, semI1)
    semG = (semG0, semG1)
    semW = (semW0, semW1)
    semC = (semC0, semC1)

    pltpu.sync_copy(zcnt.at[pl.ds(r0, STRIPE)], cnt_sh.at[pl.ds(r0, STRIPE)])

    def fill(i, _):
        onesb[pl.ds(i * 16, 16)] = jnp.ones((16,), jnp.float32)
        return 0

    lax.fori_loop(0, 7, fill, 0)
    plsc.subcore_barrier()

    def issue_idx(c, b):
        base = wid * EPW + c * CHUNK
        for j in range(RPC):
            pltpu.async_copy(edge2.at[0, pl.ds(base + j * GB, GB)],
                             sbi[b].at[j], semI[b])
            pltpu.async_copy(edge2.at[1, pl.ds(base + j * GB, GB)],
                             dbi[b].at[j], semI[b])

    def wait_idx(b):
        for j in range(RPC):
            pltpu.make_async_copy(edge2.at[0, pl.ds(0, GB)],
                                  sbi[b].at[j], semI[b]).wait()
            pltpu.make_async_copy(edge2.at[0, pl.ds(0, GB)],
                                  dbi[b].at[j], semI[b]).wait()

    def issue_gathers(b):
        for j in range(RPC):
            pltpu.async_copy(g_hbm.at[sbi[b].at[j]],
                             sr[b].at[pl.ds(j * GB, GB), :], semG[b])
            pltpu.async_copy(g_hbm.at[dbi[b].at[j]],
                             dr[b].at[pl.ds(j * GB, GB), :], semG[b])

    def wait_gathers(b):
        pltpu.make_async_copy(g_hbm.at[pl.ds(0, CHUNK), :], sr[b], semG[b]).wait()
        pltpu.make_async_copy(g_hbm.at[pl.ds(0, CHUNK), :], dr[b], semG[b]).wait()

    def wait_wb(b):
        pltpu.make_async_copy(y1_hbm.at[pl.ds(0, CHUNK), :], difb[b], semW[b]).wait()

    def wait_cnt(b):
        for j in range(RPC):
            pltpu.make_async_copy(zcnt.at[pl.ds(0, GB)],
                                  onesb.at[pl.ds(0, GB)], semC[b]).wait()

    def compute(c, b, carry):
        a1, a2 = carry

        def inner(i, carry2):
            b1, b2 = carry2
            for k in range(4):
                e = i * 4 + k
                dif = dr[b][e, :] - sr[b][e, :]
                difb[b][e, :] = dif
                b1 = b1 + dif
                b2 = b2 + dif * dif
            return (b1, b2)

        a1, a2 = lax.fori_loop(0, CHUNK // 4, inner, (a1, a2))
        pltpu.async_copy(difb[b],
                         y1_hbm.at[pl.ds(wid * EPW + c * CHUNK, CHUNK), :],
                         semW[b])
        return (a1, a2)

    issue_idx(0, 0)
    issue_idx(1, 1)
    wait_idx(0)
    issue_gathers(0)

    def half(c, b, carry):
        wait_gathers(b)
        for j in range(RPC):
            pltpu.async_copy(onesb.at[pl.ds(0, GB)],
                             cnt_sh.at[dbi[b].at[j]], semC[b], add=True)

        @pl.when(c + 1 <= NCHUNK - 1)
        def _():
            wait_idx(1 - b)
            issue_gathers(1 - b)

        @pl.when(c >= 2)
        def _():
            wait_wb(b)

        carry = compute(c, b, carry)
        wait_cnt(b)

        @pl.when(c + 2 <= NCHUNK - 1)
        def _():
            issue_idx(c + 2, b)

        return carry

    def body(c2, carry):
        c = c2 * 2
        carry = half(c, 0, carry)
        carry = half(c + 1, 1, carry)
        return carry

    z = jnp.zeros((16,), jnp.float32)
    carry = lax.fori_loop(0, (NCHUNK - 1) // 2, body, (z, z))
    a1, a2 = half(NCHUNK - 1, 0, carry)

    wait_wb(1)
    wait_wb(0)

    accv[0, :] = a1
    accv[1, :] = a2
    pltpu.sync_copy(accv, part_hbm.at[wid])

    plsc.subcore_barrier()
    pltpu.sync_copy(cnt_sh.at[pl.ds(r0, STRIPE)],
                    cnt_out.at[c_i, pl.ds(r0, STRIPE)])


def _k2(g, edge2, zcnt):
    mesh = plsc.VectorSubcoreMesh(core_axis_name="c", subcore_axis_name="s")
    f = pl.kernel(
        _k2_body,
        out_type=[
            jax.ShapeDtypeStruct((E, 16), jnp.float32),
            jax.ShapeDtypeStruct((32, 2, 16), jnp.float32),
            jax.ShapeDtypeStruct((2, NP), jnp.float32),
        ],
        mesh=mesh,
        scratch_types=[
            pltpu.VMEM((CHUNK // GB, GB), jnp.int32),
            pltpu.VMEM((CHUNK // GB, GB), jnp.int32),
            pltpu.VMEM((CHUNK // GB, GB), jnp.int32),
            pltpu.VMEM((CHUNK // GB, GB), jnp.int32),
            pltpu.VMEM((CHUNK, 16), jnp.float32),
            pltpu.VMEM((CHUNK, 16), jnp.float32),
            pltpu.VMEM((CHUNK, 16), jnp.float32),
            pltpu.VMEM((CHUNK, 16), jnp.float32),
            pltpu.VMEM((CHUNK, 16), jnp.float32),
            pltpu.VMEM((CHUNK, 16), jnp.float32),
            pltpu.VMEM((2, 16), jnp.float32),
            pltpu.VMEM((112,), jnp.float32),
            pltpu.VMEM_SHARED((NP,), jnp.float32),
            pltpu.SemaphoreType.DMA,
            pltpu.SemaphoreType.DMA,
            pltpu.SemaphoreType.DMA,
            pltpu.SemaphoreType.DMA,
            pltpu.SemaphoreType.DMA,
            pltpu.SemaphoreType.DMA,
            pltpu.SemaphoreType.DMA,
            pltpu.SemaphoreType.DMA,
        ],
        compiler_params=pltpu.CompilerParams(use_tc_tiling_on_sc=False),
    )
    return f(g, edge2, zcnt)


def _k3_body(y1_ref, p1_ref, w2_ref, bnp_ref, y2_ref, sq_ref, acc_ref):
    step = pl.program_id(0)

    @pl.when(step == 0)
    def _():
        acc_ref[...] = jnp.zeros_like(acc_ref)

    p1 = p1_ref[...]
    sums = jnp.sum(p1[:, :16], axis=0, keepdims=True)
    sqs = jnp.sum(p1[:, 16:], axis=0, keepdims=True)
    mu1 = sums / E
    var1 = sqs / E - mu1 * mu1
    g1 = bnp_ref[0:1, :]
    be1 = bnp_ref[1:2, :]
    s1 = g1 * lax.rsqrt(var1 + EPS)
    t1 = be1 - mu1 * s1

    y1 = y1_ref[...]
    segs = [jnp.maximum(y1[:, r * 16:(r + 1) * 16] * s1 + t1, 0.0)
            for r in range(8)]
    a1 = jnp.concatenate(segs, axis=1)
    z2 = jnp.dot(a1, w2_ref[...], preferred_element_type=jnp.float32)
    y2_ref[...] = z2
    acc_ref[0:1, :] += jnp.sum(z2, axis=0, keepdims=True)
    acc_ref[1:2, :] += jnp.sum(z2 * z2, axis=0, keepdims=True)

    s128 = acc_ref[0:1, :]
    q128 = acc_ref[1:2, :]
    sum16 = s128[:, 0:16]
    sq16 = q128[:, 0:16]
    for r in range(1, 8):
        sum16 = sum16 + s128[:, r * 16:(r + 1) * 16]
        sq16 = sq16 + q128[:, r * 16:(r + 1) * 16]
    mu2 = sum16 / E
    var2 = sq16 / E - mu2 * mu2
    g2 = bnp_ref[2:3, :]
    be2 = bnp_ref[3:4, :]
    s2 = g2 * lax.rsqrt(var2 + EPS)
    q2 = be2 - mu2 * s2
    sq_ref[...] = jnp.concatenate([s2, q2], axis=0)


def _k3(y1v, p1f, w2big, bnp):
    rows = E * 16 // 128
    blk = 4000
    nb = rows // blk
    return pl.pallas_call(
        _k3_body,
        grid=(nb,),
        in_specs=[
            pl.BlockSpec((blk, 128), lambda i: (i, 0)),
            pl.BlockSpec((32, 32), lambda i: (0, 0)),
            pl.BlockSpec((128, 128), lambda i: (0, 0)),
            pl.BlockSpec((4, 16), lambda i: (0, 0)),
        ],
        out_specs=[
            pl.BlockSpec((blk, 128), lambda i: (i, 0)),
            pl.BlockSpec((2, 16), lambda i: (0, 0)),
        ],
        out_shape=[
            jax.ShapeDtypeStruct((rows, 128), jnp.float32),
            jax.ShapeDtypeStruct((2, 16), jnp.float32),
        ],
        scratch_shapes=[pltpu.VMEM((8, 128), jnp.float32)],
    )(y1v, p1f, w2big, bnp)


def _k4_body(y2_hbm, edge2, s2q2, zrow, acc_out,
             acc_sh, ybuf0, ybuf1, idb0, idb1, sqv, semS0, semS1):
    c_i = lax.axis_index("c")
    s_i = lax.axis_index("s")
    wid = s_i * 2 + c_i
    r0 = s_i * STRIPE
    RPC = CHUNK // GB

    ybuf = (ybuf0, ybuf1)
    idb = (idb0, idb1)
    semS = (semS0, semS1)

    pltpu.sync_copy(zrow.at[pl.ds(r0, STRIPE), :], acc_sh.at[pl.ds(r0, STRIPE), :])
    pltpu.sync_copy(s2q2, sqv)
    plsc.subcore_barrier()

    s2 = sqv[0, :]
    q2 = sqv[1, :]

    def wait_scatter(b):
        for j in range(RPC):
            pltpu.make_async_copy(y2_hbm.at[pl.ds(0, GB), :],
                                  ybuf[b].at[pl.ds(0, GB), :], semS[b]).wait()

    def half(c, b):
        @pl.when(c >= 2)
        def _():
            wait_scatter(b)

        eb = wid * EPW + c * CHUNK
        pltpu.sync_copy(y2_hbm.at[pl.ds(eb, CHUNK), :], ybuf[b])
        for j in range(RPC):
            pltpu.sync_copy(edge2.at[1, pl.ds(eb + j * GB, GB)], idb[b].at[j])

        def inner(i, _2):
            for k in range(4):
                e = i * 4 + k
                ybuf[b][e, :] = jnp.maximum(ybuf[b][e, :] * s2 + q2, 0.0)
            return 0

        lax.fori_loop(0, CHUNK // 4, inner, 0)
        for j in range(RPC):
            pltpu.async_copy(ybuf[b].at[pl.ds(j * GB, GB), :],
                             acc_sh.at[idb[b].at[j]], semS[b], add=True)

    def body(c2, _):
        c = c2 * 2
        half(c, 0)
        half(c + 1, 1)
        return 0

    lax.fori_loop(0, (NCHUNK - 1) // 2, body, 0)
    half(NCHUNK - 1, 0)
    wait_scatter(1)
    wait_scatter(0)

    plsc.subcore_barrier()
    pltpu.sync_copy(acc_sh.at[pl.ds(r0, STRIPE), :],
                    acc_out.at[c_i, pl.ds(r0, STRIPE), :])


def _k4(y2, edge2, s2q2, zrow):
    mesh = plsc.VectorSubcoreMesh(core_axis_name="c", subcore_axis_name="s")
    f = pl.kernel(
        _k4_body,
        out_type=jax.ShapeDtypeStruct((2, NP, 16), jnp.float32),
        mesh=mesh,
        scratch_types=[
            pltpu.VMEM_SHARED((NP, 16), jnp.float32),
            pltpu.VMEM((CHUNK, 16), jnp.float32),
            pltpu.VMEM((CHUNK, 16), jnp.float32),
            pltpu.VMEM((CHUNK // GB, GB), jnp.int32),
            pltpu.VMEM((CHUNK // GB, GB), jnp.int32),
            pltpu.VMEM((2, 16), jnp.float32),
            pltpu.SemaphoreType.DMA,
            pltpu.SemaphoreType.DMA,
        ],
        compiler_params=pltpu.CompilerParams(use_tc_tiling_on_sc=False),
    )
    return f(y2, edge2, s2q2, zrow)


_BLK5 = _NRP // 4


def _fold8(v2):
    out = v2[:, 0:16]
    for r in range(1, 8):
        out = out + v2[:, r * 16:(r + 1) * 16]
    return out


def _tile8(v):
    return jnp.concatenate([v] * 8, axis=1)


def _k5a_body(p_ref, v_ref, a0_ref, a1_ref, cb_ref, wap_ref, wav_ref, wb_ref,
              z_ref, st_ref, acc_ref):
    step = pl.program_id(0)

    @pl.when(step == 0)
    def _():
        acc_ref[...] = jnp.zeros_like(acc_ref)

    accs = a0_ref[...] + a1_ref[...]
    inv = 1.0 / jnp.maximum(cb_ref[...], 1.0)
    lane = lax.broadcasted_iota(jnp.int32, accs.shape, 1) % 16
    aggr = jnp.where(lane < 2, accs, accs * inv)
    z = (jnp.dot(p_ref[...], wap_ref[...], preferred_element_type=jnp.float32)
         + jnp.dot(v_ref[...], wav_ref[...], preferred_element_type=jnp.float32)
         + jnp.dot(aggr, wb_ref[...], preferred_element_type=jnp.float32))
    z_ref[...] = z
    acc_ref[0:1, :] += jnp.sum(z, axis=0, keepdims=True)
    acc_ref[1:2, :] += jnp.sum(z * z, axis=0, keepdims=True)
    st_ref[...] = acc_ref[...]


def _k5a(pos64p, vel64p, accall, cntb, wap, wav, wb):
    return pl.pallas_call(
        _k5a_body,
        grid=(4,),
        in_specs=[
            pl.BlockSpec((_BLK5, 64), lambda i: (i, 0)),
            pl.BlockSpec((_BLK5, 64), lambda i: (i, 0)),
            pl.BlockSpec((_BLK5, 128), lambda i: (i, 0)),
            pl.BlockSpec((_BLK5, 128), lambda i: (i + 4, 0)),
            pl.BlockSpec((_BLK5, 128), lambda i: (i, 0)),
            pl.BlockSpec((64, 128), lambda i: (0, 0)),
            pl.BlockSpec((64, 128), lambda i: (0, 0)),
            pl.BlockSpec((128, 128), lambda i: (0, 0)),
        ],
        out_specs=[
            pl.BlockSpec((_BLK5, 128), lambda i: (i, 0)),
            pl.BlockSpec((2, 128), lambda i: (0, 0)),
        ],
        out_shape=[
            jax.ShapeDtypeStruct((_NRP, 128), jnp.float32),
            jax.ShapeDtypeStruct((2, 128), jnp.float32),
        ],
        scratch_shapes=[pltpu.VMEM((2, 128), jnp.float32)],
    )(pos64p, vel64p, accall, accall, cntb, wap, wav, wb)


def _k5b_body(z_ref, st_ref, bn_ref, w2_ref, z2_ref, st2_ref, acc_ref):
    step = pl.program_id(0)

    @pl.when(step == 0)
    def _():
        acc_ref[...] = jnp.zeros_like(acc_ref)

    st = _fold8(st_ref[...])
    mu = st[0:1, :] / N
    var = st[1:2, :] / N - mu * mu
    s = bn_ref[0:1, :] * lax.rsqrt(var + EPS)
    t = bn_ref[1:2, :] - mu * s
    a = jnp.maximum(z_ref[...] * _tile8(s) + _tile8(t), 0.0)
    z2 = jnp.dot(a, w2_ref[...], preferred_element_type=jnp.float32)
    rowid = lax.broadcasted_iota(jnp.int32, z2.shape, 0) + step * _BLK5
    z2 = jnp.where(rowid < _NR, z2, 0.0)
    z2_ref[...] = z2
    acc_ref[0:1, :] += jnp.sum(z2, axis=0, keepdims=True)
    acc_ref[1:2, :] += jnp.sum(z2 * z2, axis=0, keepdims=True)
    st2_ref[...] = acc_ref[...]


def _k5b(z1, st1, ubn, w2u):
    return pl.pallas_call(
        _k5b_body,
        grid=(4,),
        in_specs=[
            pl.BlockSpec((_BLK5, 128), lambda i: (i, 0)),
            pl.BlockSpec((2, 128), lambda i: (0, 0)),
            pl.BlockSpec((4, 16), lambda i: (0, 0)),
            pl.BlockSpec((128, 128), lambda i: (0, 0)),
        ],
        out_specs=[
            pl.BlockSpec((_BLK5, 128), lambda i: (i, 0)),
            pl.BlockSpec((2, 128), lambda i: (0, 0)),
        ],
        out_shape=[
            jax.ShapeDtypeStruct((_NRP, 128), jnp.float32),
            jax.ShapeDtypeStruct((2, 128), jnp.float32),
        ],
        scratch_shapes=[pltpu.VMEM((2, 128), jnp.float32)],
    )(z1, st1, ubn, w2u)


def _k5c_body(z_ref, st_ref, bn_ref, pw_ref, pb_ref, out_ref):
    st = _fold8(st_ref[...])
    mu = st[0:1, :] / N
    var = st[1:2, :] / N - mu * mu
    s = bn_ref[2:3, :] * lax.rsqrt(var + EPS)
    t = bn_ref[3:4, :] - mu * s
    a = jnp.maximum(z_ref[...] * _tile8(s) + _tile8(t), 0.0)
    out_ref[...] = jnp.dot(a, pw_ref[...],
                           preferred_element_type=jnp.float32) + pb_ref[...]


def _k5c(z2, st2, ubn, pwb, pbt):
    return pl.pallas_call(
        _k5c_body,
        grid=(4,),
        in_specs=[
            pl.BlockSpec((_BLK5, 128), lambda i: (i, 0)),
            pl.BlockSpec((2, 128), lambda i: (0, 0)),
            pl.BlockSpec((4, 16), lambda i: (0, 0)),
            pl.BlockSpec((128, 16), lambda i: (0, 0)),
            pl.BlockSpec((1, 16), lambda i: (0, 0)),
        ],
        out_specs=pl.BlockSpec((_BLK5, 16), lambda i: (i, 0)),
        out_shape=jax.ShapeDtypeStruct((_NRP, 16), jnp.float32),
    )(z2, st2, ubn, pwb, pbt)


def kernel(pos, vel, edge_index, mW1, mb1, mg1, mbe1, mW2, mb2, mg2, mbe2,
           uW1, ub1, ug1, ube1, uW2, ub2, ug2, ube2, pW, pb):
    eye8 = jnp.eye(8, dtype=jnp.float32)
    pos64 = pos.reshape(_NR, 64)
    vel64 = vel.reshape(_NR, 64)
    g128 = _k1(pos64, vel64,
               jnp.kron(eye8, mW1[:8, :]), jnp.kron(eye8, mW1[8:, :]))
    g = g128.reshape(N, 16)

    zcnt = jnp.zeros((NP,), jnp.float32)
    y1, part1, cntp = _k2(g, edge_index, zcnt)

    y1v = y1.reshape(E * 16 // 128, 128)
    p1f = part1.reshape(32, 32)
    w2big = jnp.kron(eye8, mW2)
    bnp = jnp.stack([mg1, mbe1, mg2, mbe2])
    y2v, s2q2 = _k3(y1v, p1f, w2big, bnp)
    y2 = y2v.reshape(E, 16)

    zrow = jnp.zeros((NP, 16), jnp.float32)
    accp = _k4(y2, edge_index, s2q2, zrow)

    ubn = jnp.stack([ug1, ube1, ug2, ube2])
    zpad = jnp.zeros((_NRP - _NR, 64), jnp.float32)
    pos64p = jnp.concatenate([pos64, zpad], axis=0)
    vel64p = jnp.concatenate([vel64, zpad], axis=0)
    accall = accp.reshape(2 * _NRP, 128)
    cnt_t = cntp[0] + cntp[1]
    cntb = jnp.repeat(cnt_t, 16).reshape(_NRP, 128)
    wap = jnp.kron(eye8, uW1[:8, :])
    wav = jnp.kron(eye8, uW1[8:16, :])
    wb = jnp.kron(eye8, uW1[16:, :])
    z1, st1 = _k5a(pos64p, vel64p, accall, cntb, wap, wav, wb)
    z2, st2 = _k5b(z1, st1, ubn, jnp.kron(eye8, uW2))
    out128 = _k5c(z2, st2, ubn, jnp.kron(eye8, pW), jnp.tile(pb, 8).reshape(1, 16))
    return out128[:_NR].reshape(N, 2)


# speedup vs baseline: 1.4102x; 1.4102x over previous
import functools

import jax
import jax.numpy as jnp
from jax import lax
from jax.experimental import pallas as pl
from jax.experimental.pallas import tpu as pltpu
from jax.experimental.pallas import tpu_sc as plsc

N = 100000
E = 3200000
NP = 100096
STRIPE = 6256
EPW = E // 32
CHUNK = 800
NCHUNK = EPW // CHUNK
GB = 80
EPS = 1e-5


_NR = N * 16 // 128
_NRP = NP * 16 // 128


def _k1_body(p_ref, v_ref, wp_ref, wv_ref, g_ref):
    g_ref[...] = (
        jnp.dot(p_ref[...], wp_ref[...], preferred_element_type=jnp.float32)
        + jnp.dot(v_ref[...], wv_ref[...], preferred_element_type=jnp.float32))


def _k1(pos64, vel64, wp, wv):
    return pl.pallas_call(
        _k1_body,
        out_shape=jax.ShapeDtypeStruct((_NR, 128), jnp.float32),
    )(pos64, vel64, wp, wv)


def _k2_body(g_hbm, edge2, zcnt, y1_hbm, part_hbm, cnt_out,
             sbi0, dbi0, sbi1, dbi1, sr0, dr0, sr1, dr1, difb0, difb1,
             accv, onesb, cnt_sh,
             semI0, semI1, semG0, semG1, semW0, semW1, semC0, semC1):
    c_i = lax.axis_index("c")
    s_i = lax.axis_index("s")
    wid = s_i * 2 + c_i
    r0 = s_i * STRIPE
    RPC = CHUNK // GB

    sbi = (sbi0, sbi1)
    dbi = (dbi0, dbi1)
    sr = (sr0, sr1)
    dr = (dr0, dr1)
    difb = (difb0, difb1)
    semI = (semI0, semI1)
    semG = (semG0, semG1)
    semW = (semW0, semW1)
    semC = (semC0, semC1)

    pltpu.sync_copy(zcnt.at[pl.ds(r0, STRIPE)], cnt_sh.at[pl.ds(r0, STRIPE)])

    def fill(i, _):
        onesb[pl.ds(i * 16, 16)] = jnp.ones((16,), jnp.float32)
        return 0

    lax.fori_loop(0, 7, fill, 0)
    plsc.subcore_barrier()

    def issue_idx(c, b):
        base = wid * EPW + c * CHUNK
        for j in range(RPC):
            pltpu.async_copy(edge2.at[0, pl.ds(base + j * GB, GB)],
                             sbi[b].at[j], semI[b])
            pltpu.async_copy(edge2.at[1, pl.ds(base + j * GB, GB)],
                             dbi[b].at[j], semI[b])

    def wait_idx(b):
        for j in range(RPC):
            pltpu.make_async_copy(edge2.at[0, pl.ds(0, GB)],
                                  sbi[b].at[j], semI[b]).wait()
            pltpu.make_async_copy(edge2.at[0, pl.ds(0, GB)],
                                  dbi[b].at[j], semI[b]).wait()

    def issue_gathers(b):
        for j in range(RPC):
            pltpu.async_copy(g_hbm.at[sbi[b].at[j]],
                             sr[b].at[pl.ds(j * GB, GB), :], semG[b])
            pltpu.async_copy(g_hbm.at[dbi[b].at[j]],
                             dr[b].at[pl.ds(j * GB, GB), :], semG[b])

    def wait_gathers(b):
        pltpu.make_async_copy(g_hbm.at[pl.ds(0, CHUNK), :], sr[b], semG[b]).wait()
        pltpu.make_async_copy(g_hbm.at[pl.ds(0, CHUNK), :], dr[b], semG[b]).wait()

    def wait_wb(b):
        pltpu.make_async_copy(y1_hbm.at[pl.ds(0, CHUNK), :], difb[b], semW[b]).wait()

    def wait_cnt(b):
        for j in range(RPC):
            pltpu.make_async_copy(zcnt.at[pl.ds(0, GB)],
                                  onesb.at[pl.ds(0, GB)], semC[b]).wait()

    def compute(c, b, carry):
        a1, a2 = carry

        def inner(i, carry2):
            b1, b2 = carry2
            for k in range(4):
                e = i * 4 + k
                dif = dr[b][e, :] - sr[b][e, :]
                difb[b][e, :] = dif
                b1 = b1 + dif
                b2 = b2 + dif * dif
            return (b1, b2)

        a1, a2 = lax.fori_loop(0, CHUNK // 4, inner, (a1, a2))
        pltpu.async_copy(difb[b],
                         y1_hbm.at[pl.ds(wid * EPW + c * CHUNK, CHUNK), :],
                         semW[b])
        return (a1, a2)

    issue_idx(0, 0)
    issue_idx(1, 1)
    wait_idx(0)
    issue_gathers(0)

    def half(c, b, carry):
        wait_gathers(b)
        for j in range(RPC):
            pltpu.async_copy(onesb.at[pl.ds(0, GB)],
                             cnt_sh.at[dbi[b].at[j]], semC[b], add=True)

        @pl.when(c + 1 <= NCHUNK - 1)
        def _():
            wait_idx(1 - b)
            issue_gathers(1 - b)

        @pl.when(c >= 2)
        def _():
            wait_wb(b)

        carry = compute(c, b, carry)
        wait_cnt(b)

        @pl.when(c + 2 <= NCHUNK - 1)
        def _():
            issue_idx(c + 2, b)

        return carry

    def body(c2, carry):
        c = c2 * 2
        carry = half(c, 0, carry)
        carry = half(c + 1, 1, carry)
        return carry

    z = jnp.zeros((16,), jnp.float32)
    carry = lax.fori_loop(0, (NCHUNK - 1) // 2, body, (z, z))
    a1, a2 = half(NCHUNK - 1, 0, carry)

    wait_wb(1)
    wait_wb(0)

    accv[0, :] = a1
    accv[1, :] = a2
    pltpu.sync_copy(accv, part_hbm.at[wid])

    plsc.subcore_barrier()
    pltpu.sync_copy(cnt_sh.at[pl.ds(r0, STRIPE)],
                    cnt_out.at[c_i, pl.ds(r0, STRIPE)])


def _k2(g, edge2, zcnt):
    mesh = plsc.VectorSubcoreMesh(core_axis_name="c", subcore_axis_name="s")
    f = pl.kernel(
        _k2_body,
        out_type=[
            jax.ShapeDtypeStruct((E, 16), jnp.float32),
            jax.ShapeDtypeStruct((32, 2, 16), jnp.float32),
            jax.ShapeDtypeStruct((2, NP), jnp.float32),
        ],
        mesh=mesh,
        scratch_types=[
            pltpu.VMEM((CHUNK // GB, GB), jnp.int32),
            pltpu.VMEM((CHUNK // GB, GB), jnp.int32),
            pltpu.VMEM((CHUNK // GB, GB), jnp.int32),
            pltpu.VMEM((CHUNK // GB, GB), jnp.int32),
            pltpu.VMEM((CHUNK, 16), jnp.float32),
            pltpu.VMEM((CHUNK, 16), jnp.float32),
            pltpu.VMEM((CHUNK, 16), jnp.float32),
            pltpu.VMEM((CHUNK, 16), jnp.float32),
            pltpu.VMEM((CHUNK, 16), jnp.float32),
            pltpu.VMEM((CHUNK, 16), jnp.float32),
            pltpu.VMEM((2, 16), jnp.float32),
            pltpu.VMEM((112,), jnp.float32),
            pltpu.VMEM_SHARED((NP,), jnp.float32),
            pltpu.SemaphoreType.DMA,
            pltpu.SemaphoreType.DMA,
            pltpu.SemaphoreType.DMA,
            pltpu.SemaphoreType.DMA,
            pltpu.SemaphoreType.DMA,
            pltpu.SemaphoreType.DMA,
            pltpu.SemaphoreType.DMA,
            pltpu.SemaphoreType.DMA,
        ],
        compiler_params=pltpu.CompilerParams(use_tc_tiling_on_sc=False),
    )
    return f(g, edge2, zcnt)


def _k3_body(y1_ref, p1_ref, w2_ref, bnp_ref, y2_ref, sq_ref, acc_ref):
    step = pl.program_id(0)

    @pl.when(step == 0)
    def _():
        acc_ref[...] = jnp.zeros_like(acc_ref)

    p1 = p1_ref[...]
    sums = jnp.sum(p1[:, :16], axis=0, keepdims=True)
    sqs = jnp.sum(p1[:, 16:], axis=0, keepdims=True)
    mu1 = sums / E
    var1 = sqs / E - mu1 * mu1
    g1 = bnp_ref[0:1, :]
    be1 = bnp_ref[1:2, :]
    s1 = g1 * lax.rsqrt(var1 + EPS)
    t1 = be1 - mu1 * s1

    y1 = y1_ref[...]
    segs = [jnp.maximum(y1[:, r * 16:(r + 1) * 16] * s1 + t1, 0.0)
            for r in range(8)]
    a1 = jnp.concatenate(segs, axis=1)
    z2 = jnp.dot(a1, w2_ref[...], preferred_element_type=jnp.float32)
    y2_ref[...] = z2
    acc_ref[0:1, :] += jnp.sum(z2, axis=0, keepdims=True)
    acc_ref[1:2, :] += jnp.sum(z2 * z2, axis=0, keepdims=True)

    s128 = acc_ref[0:1, :]
    q128 = acc_ref[1:2, :]
    sum16 = s128[:, 0:16]
    sq16 = q128[:, 0:16]
    for r in range(1, 8):
        sum16 = sum16 + s128[:, r * 16:(r + 1) * 16]
        sq16 = sq16 + q128[:, r * 16:(r + 1) * 16]
    mu2 = sum16 / E
    var2 = sq16 / E - mu2 * mu2
    g2 = bnp_ref[2:3, :]
    be2 = bnp_ref[3:4, :]
    s2 = g2 * lax.rsqrt(var2 + EPS)
    q2 = be2 - mu2 * s2
    sq_ref[...] = jnp.concatenate([s2, q2], axis=0)


def _k3(y1v, p1f, w2big, bnp):
    rows = E * 16 // 128
    blk = 4000
    nb = rows // blk
    return pl.pallas_call(
        _k3_body,
        grid=(nb,),
        in_specs=[
            pl.BlockSpec((blk, 128), lambda i: (i, 0)),
            pl.BlockSpec((32, 32), lambda i: (0, 0)),
            pl.BlockSpec((128, 128), lambda i: (0, 0)),
            pl.BlockSpec((4, 16), lambda i: (0, 0)),
        ],
        out_specs=[
            pl.BlockSpec((blk, 128), lambda i: (i, 0)),
            pl.BlockSpec((2, 16), lambda i: (0, 0)),
        ],
        out_shape=[
            jax.ShapeDtypeStruct((rows, 128), jnp.float32),
            jax.ShapeDtypeStruct((2, 16), jnp.float32),
        ],
        scratch_shapes=[pltpu.VMEM((8, 128), jnp.float32)],
    )(y1v, p1f, w2big, bnp)


def _k4_body(y2_hbm, edge2, s2q2, zrow, acc_out,
             acc_sh, ybuf0, ybuf1, idb0, idb1, sqv, semS0, semS1, semR):
    c_i = lax.axis_index("c")
    s_i = lax.axis_index("s")
    wid = s_i * 2 + c_i
    r0 = s_i * STRIPE
    RPC = CHUNK // GB

    ybuf = (ybuf0, ybuf1)
    idb = (idb0, idb1)
    semS = (semS0, semS1)

    pltpu.sync_copy(zrow.at[pl.ds(r0, STRIPE), :], acc_sh.at[pl.ds(r0, STRIPE), :])
    pltpu.sync_copy(s2q2, sqv)
    plsc.subcore_barrier()

    s2 = sqv[0, :]
    q2 = sqv[1, :]

    def wait_scatter(b):
        for j in range(RPC):
            pltpu.make_async_copy(y2_hbm.at[pl.ds(0, GB), :],
                                  ybuf[b].at[pl.ds(0, GB), :], semS[b]).wait()

    def half(c, b):
        @pl.when(c >= 2)
        def _():
            wait_scatter(b)

        eb = wid * EPW + c * CHUNK
        cps = [pltpu.async_copy(y2_hbm.at[pl.ds(eb, CHUNK), :], ybuf[b], semR)]
        for j in range(RPC):
            cps.append(pltpu.async_copy(
                edge2.at[1, pl.ds(eb + j * GB, GB)], idb[b].at[j], semR))
        for cp in cps:
            cp.wait()

        def inner(i, _2):
            for k in range(4):
                e = i * 4 + k
                ybuf[b][e, :] = jnp.maximum(ybuf[b][e, :] * s2 + q2, 0.0)
            return 0

        lax.fori_loop(0, CHUNK // 4, inner, 0)
        for j in range(RPC):
            pltpu.async_copy(ybuf[b].at[pl.ds(j * GB, GB), :],
                             acc_sh.at[idb[b].at[j]], semS[b], add=True)

    def body(c2, _):
        c = c2 * 2
        half(c, 0)
        half(c + 1, 1)
        return 0

    lax.fori_loop(0, (NCHUNK - 1) // 2, body, 0)
    half(NCHUNK - 1, 0)
    wait_scatter(1)
    wait_scatter(0)

    plsc.subcore_barrier()
    pltpu.sync_copy(acc_sh.at[pl.ds(r0, STRIPE), :],
                    acc_out.at[c_i, pl.ds(r0, STRIPE), :])


def _k4(y2, edge2, s2q2, zrow):
    mesh = plsc.VectorSubcoreMesh(core_axis_name="c", subcore_axis_name="s")
    f = pl.kernel(
        _k4_body,
        out_type=jax.ShapeDtypeStruct((2, NP, 16), jnp.float32),
        mesh=mesh,
        scratch_types=[
            pltpu.VMEM_SHARED((NP, 16), jnp.float32),
            pltpu.VMEM((CHUNK, 16), jnp.float32),
            pltpu.VMEM((CHUNK, 16), jnp.float32),
            pltpu.VMEM((CHUNK // GB, GB), jnp.int32),
            pltpu.VMEM((CHUNK // GB, GB), jnp.int32),
            pltpu.VMEM((2, 16), jnp.float32),
            pltpu.SemaphoreType.DMA,
            pltpu.SemaphoreType.DMA,
            pltpu.SemaphoreType.DMA,
        ],
        compiler_params=pltpu.CompilerParams(use_tc_tiling_on_sc=False),
    )
    return f(y2, edge2, s2q2, zrow)


_BLK5 = _NRP // 4


def _fold8(v2):
    out = v2[:, 0:16]
    for r in range(1, 8):
        out = out + v2[:, r * 16:(r + 1) * 16]
    return out


def _tile8(v):
    return jnp.concatenate([v] * 8, axis=1)


def _k5a_body(p_ref, v_ref, a0_ref, a1_ref, cb_ref, wap_ref, wav_ref, wb_ref,
              z_ref, st_ref, acc_ref):
    step = pl.program_id(0)

    @pl.when(step == 0)
    def _():
        acc_ref[...] = jnp.zeros_like(acc_ref)

    accs = a0_ref[...] + a1_ref[...]
    inv = 1.0 / jnp.maximum(cb_ref[...], 1.0)
    lane = lax.broadcasted_iota(jnp.int32, accs.shape, 1) % 16
    aggr = jnp.where(lane < 2, accs, accs * inv)
    z = (jnp.dot(p_ref[...], wap_ref[...], preferred_element_type=jnp.float32)
         + jnp.dot(v_ref[...], wav_ref[...], preferred_element_type=jnp.float32)
         + jnp.dot(aggr, wb_ref[...], preferred_element_type=jnp.float32))
    z_ref[...] = z
    acc_ref[0:1, :] += jnp.sum(z, axis=0, keepdims=True)
    acc_ref[1:2, :] += jnp.sum(z * z, axis=0, keepdims=True)
    st_ref[...] = acc_ref[...]


def _k5a(pos64p, vel64p, accall, cntb, wap, wav, wb):
    return pl.pallas_call(
        _k5a_body,
        grid=(4,),
        in_specs=[
            pl.BlockSpec((_BLK5, 64), lambda i: (i, 0)),
            pl.BlockSpec((_BLK5, 64), lambda i: (i, 0)),
            pl.BlockSpec((_BLK5, 128), lambda i: (i, 0)),
            pl.BlockSpec((_BLK5, 128), lambda i: (i + 4, 0)),
            pl.BlockSpec((_BLK5, 128), lambda i: (i, 0)),
            pl.BlockSpec((64, 128), lambda i: (0, 0)),
            pl.BlockSpec((64, 128), lambda i: (0, 0)),
            pl.BlockSpec((128, 128), lambda i: (0, 0)),
        ],
        out_specs=[
            pl.BlockSpec((_BLK5, 128), lambda i: (i, 0)),
            pl.BlockSpec((2, 128), lambda i: (0, 0)),
        ],
        out_shape=[
            jax.ShapeDtypeStruct((_NRP, 128), jnp.float32),
            jax.ShapeDtypeStruct((2, 128), jnp.float32),
        ],
        scratch_shapes=[pltpu.VMEM((2, 128), jnp.float32)],
    )(pos64p, vel64p, accall, accall, cntb, wap, wav, wb)


def _k5b_body(z_ref, st_ref, bn_ref, w2_ref, z2_ref, st2_ref, acc_ref):
    step = pl.program_id(0)

    @pl.when(step == 0)
    def _():
        acc_ref[...] = jnp.zeros_like(acc_ref)

    st = _fold8(st_ref[...])
    mu = st[0:1, :] / N
    var = st[1:2, :] / N - mu * mu
    s = bn_ref[0:1, :] * lax.rsqrt(var + EPS)
    t = bn_ref[1:2, :] - mu * s
    a = jnp.maximum(z_ref[...] * _tile8(s) + _tile8(t), 0.0)
    z2 = jnp.dot(a, w2_ref[...], preferred_element_type=jnp.float32)
    rowid = lax.broadcasted_iota(jnp.int32, z2.shape, 0) + step * _BLK5
    z2 = jnp.where(rowid < _NR, z2, 0.0)
    z2_ref[...] = z2
    acc_ref[0:1, :] += jnp.sum(z2, axis=0, keepdims=True)
    acc_ref[1:2, :] += jnp.sum(z2 * z2, axis=0, keepdims=True)
    st2_ref[...] = acc_ref[...]


def _k5b(z1, st1, ubn, w2u):
    return pl.pallas_call(
        _k5b_body,
        grid=(4,),
        in_specs=[
            pl.BlockSpec((_BLK5, 128), lambda i: (i, 0)),
            pl.BlockSpec((2, 128), lambda i: (0, 0)),
            pl.BlockSpec((4, 16), lambda i: (0, 0)),
            pl.BlockSpec((128, 128), lambda i: (0, 0)),
        ],
        out_specs=[
            pl.BlockSpec((_BLK5, 128), lambda i: (i, 0)),
            pl.BlockSpec((2, 128), lambda i: (0, 0)),
        ],
        out_shape=[
            jax.ShapeDtypeStruct((_NRP, 128), jnp.float32),
            jax.ShapeDtypeStruct((2, 128), jnp.float32),
        ],
        scratch_shapes=[pltpu.VMEM((2, 128), jnp.float32)],
    )(z1, st1, ubn, w2u)


def _k5c_body(z_ref, st_ref, bn_ref, pw_ref, pb_ref, out_ref):
    st = _fold8(st_ref[...])
    mu = st[0:1, :] / N
    var = st[1:2, :] / N - mu * mu
    s = bn_ref[2:3, :] * lax.rsqrt(var + EPS)
    t = bn_ref[3:4, :] - mu * s
    a = jnp.maximum(z_ref[...] * _tile8(s) + _tile8(t), 0.0)
    out_ref[...] = jnp.dot(a, pw_ref[...],
                           preferred_element_type=jnp.float32) + pb_ref[...]


def _k5c(z2, st2, ubn, pwb, pbt):
    return pl.pallas_call(
        _k5c_body,
        grid=(4,),
        in_specs=[
            pl.BlockSpec((_BLK5, 128), lambda i: (i, 0)),
            pl.BlockSpec((2, 128), lambda i: (0, 0)),
            pl.BlockSpec((4, 16), lambda i: (0, 0)),
            pl.BlockSpec((128, 16), lambda i: (0, 0)),
            pl.BlockSpec((1, 16), lambda i: (0, 0)),
        ],
        out_specs=pl.BlockSpec((_BLK5, 16), lambda i: (i, 0)),
        out_shape=jax.ShapeDtypeStruct((_NRP, 16), jnp.float32),
    )(z2, st2, ubn, pwb, pbt)


def kernel(pos, vel, edge_index, mW1, mb1, mg1, mbe1, mW2, mb2, mg2, mbe2,
           uW1, ub1, ug1, ube1, uW2, ub2, ug2, ube2, pW, pb):
    eye8 = jnp.eye(8, dtype=jnp.float32)
    pos64 = pos.reshape(_NR, 64)
    vel64 = vel.reshape(_NR, 64)
    g128 = _k1(pos64, vel64,
               jnp.kron(eye8, mW1[:8, :]), jnp.kron(eye8, mW1[8:, :]))
    g = g128.reshape(N, 16)

    zcnt = jnp.zeros((NP,), jnp.float32)
    y1, part1, cntp = _k2(g, edge_index, zcnt)

    y1v = y1.reshape(E * 16 // 128, 128)
    p1f = part1.reshape(32, 32)
    w2big = jnp.kron(eye8, mW2)
    bnp = jnp.stack([mg1, mbe1, mg2, mbe2])
    y2v, s2q2 = _k3(y1v, p1f, w2big, bnp)
    y2 = y2v.reshape(E, 16)

    zrow = jnp.zeros((NP, 16), jnp.float32)
    accp = _k4(y2, edge_index, s2q2, zrow)

    ubn = jnp.stack([ug1, ube1, ug2, ube2])
    zpad = jnp.zeros((_NRP - _NR, 64), jnp.float32)
    pos64p = jnp.concatenate([pos64, zpad], axis=0)
    vel64p = jnp.concatenate([vel64, zpad], axis=0)
    accall = accp.reshape(2 * _NRP, 128)
    cnt_t = cntp[0] + cntp[1]
    cntb = jnp.repeat(cnt_t, 16).reshape(_NRP, 128)
    wap = jnp.kron(eye8, uW1[:8, :])
    wav = jnp.kron(eye8, uW1[8:16, :])
    wb = jnp.kron(eye8, uW1[16:, :])
    z1, st1 = _k5a(pos64p, vel64p, accall, cntb, wap, wav, wb)
    z2, st2 = _k5b(z1, st1, ubn, jnp.kron(eye8, uW2))
    out128 = _k5c(z2, st2, ubn, jnp.kron(eye8, pW), jnp.tile(pb, 8).reshape(1, 16))
    return out128[:_NR].reshape(N, 2)


# speedup vs baseline: 1.4206x; 1.0074x over previous
import functools

import jax
import jax.numpy as jnp
from jax import lax
from jax.experimental import pallas as pl
from jax.experimental.pallas import tpu as pltpu
from jax.experimental.pallas import tpu_sc as plsc

N = 100000
E = 3200000
NP = 100096
STRIPE = 6256
EPW = E // 32
CHUNK = 800
NCHUNK = EPW // CHUNK
GB = 80
EPS = 1e-5


_NR = N * 16 // 128
_NRP = NP * 16 // 128


def _k1_body(p_ref, v_ref, wp_ref, wv_ref, g_ref):
    g_ref[...] = (
        jnp.dot(p_ref[...], wp_ref[...], preferred_element_type=jnp.float32)
        + jnp.dot(v_ref[...], wv_ref[...], preferred_element_type=jnp.float32))


def _k1(pos64, vel64, wp, wv):
    return pl.pallas_call(
        _k1_body,
        out_shape=jax.ShapeDtypeStruct((_NR, 128), jnp.float32),
    )(pos64, vel64, wp, wv)


def _k2_body(g_hbm, edge2, zcnt, y1_hbm, part_hbm, cnt_out,
             sbi0, dbi0, sbi1, dbi1, sr0, dr0, sr1, dr1, difb0, difb1,
             accv, onesb, cnt_sh,
             semI0, semI1, semG0, semG1, semW0, semW1, semC0, semC1):
    c_i = lax.axis_index("c")
    s_i = lax.axis_index("s")
    wid = s_i * 2 + c_i
    r0 = s_i * STRIPE
    RPC = CHUNK // GB

    sbi = (sbi0, sbi1)
    dbi = (dbi0, dbi1)
    sr = (sr0, sr1)
    dr = (dr0, dr1)
    difb = (difb0, difb1)
    semI = (semI0, semI1)
    semG = (semG0, semG1)
    semW = (semW0, semW1)
    semC = (semC0, semC1)

    pltpu.sync_copy(zcnt.at[pl.ds(r0, STRIPE)], cnt_sh.at[pl.ds(r0, STRIPE)])

    def fill(i, _):
        onesb[pl.ds(i * 16, 16)] = jnp.ones((16,), jnp.float32)
        return 0

    lax.fori_loop(0, 7, fill, 0)
    plsc.subcore_barrier()

    def issue_idx(c, b):
        base = wid * EPW + c * CHUNK
        for j in range(RPC):
            pltpu.async_copy(edge2.at[0, pl.ds(base + j * GB, GB)],
                             sbi[b].at[j], semI[b])
            pltpu.async_copy(edge2.at[1, pl.ds(base + j * GB, GB)],
                             dbi[b].at[j], semI[b])

    def wait_idx(b):
        for j in range(RPC):
            pltpu.make_async_copy(edge2.at[0, pl.ds(0, GB)],
                                  sbi[b].at[j], semI[b]).wait()
            pltpu.make_async_copy(edge2.at[0, pl.ds(0, GB)],
                                  dbi[b].at[j], semI[b]).wait()

    def issue_gathers(b):
        for j in range(RPC):
            pltpu.async_copy(g_hbm.at[sbi[b].at[j]],
                             sr[b].at[pl.ds(j * GB, GB), :], semG[b])
            pltpu.async_copy(g_hbm.at[dbi[b].at[j]],
                             dr[b].at[pl.ds(j * GB, GB), :], semG[b])

    def wait_gathers(b):
        pltpu.make_async_copy(g_hbm.at[pl.ds(0, CHUNK), :], sr[b], semG[b]).wait()
        pltpu.make_async_copy(g_hbm.at[pl.ds(0, CHUNK), :], dr[b], semG[b]).wait()

    def wait_wb(b):
        pltpu.make_async_copy(y1_hbm.at[pl.ds(0, CHUNK), :], difb[b], semW[b]).wait()

    def wait_cnt(b):
        for j in range(RPC):
            pltpu.make_async_copy(zcnt.at[pl.ds(0, GB)],
                                  onesb.at[pl.ds(0, GB)], semC[b]).wait()

    def compute(c, b, carry):
        a1, a2 = carry

        def inner(i, carry2):
            b1, b2 = carry2
            for k in range(4):
                e = i * 4 + k
                dif = dr[b][e, :] - sr[b][e, :]
                difb[b][e, :] = dif
                b1 = b1 + dif
                b2 = b2 + dif * dif
            return (b1, b2)

        a1, a2 = lax.fori_loop(0, CHUNK // 4, inner, (a1, a2))
        pltpu.async_copy(difb[b],
                         y1_hbm.at[pl.ds(wid * EPW + c * CHUNK, CHUNK), :],
                         semW[b])
        return (a1, a2)

    issue_idx(0, 0)
    issue_idx(1, 1)
    wait_idx(0)
    issue_gathers(0)

    def half(c, b, carry):
        wait_gathers(b)
        for j in range(RPC):
            pltpu.async_copy(onesb.at[pl.ds(0, GB)],
                             cnt_sh.at[dbi[b].at[j]], semC[b], add=True)

        @pl.when(c + 1 <= NCHUNK - 1)
        def _():
            wait_idx(1 - b)
            issue_gathers(1 - b)

        @pl.when(c >= 2)
        def _():
            wait_wb(b)

        carry = compute(c, b, carry)
        wait_cnt(b)

        @pl.when(c + 2 <= NCHUNK - 1)
        def _():
            issue_idx(c + 2, b)

        return carry

    def body(c2, carry):
        c = c2 * 2
        carry = half(c, 0, carry)
        carry = half(c + 1, 1, carry)
        return carry

    z = jnp.zeros((16,), jnp.float32)
    carry = lax.fori_loop(0, (NCHUNK - 1) // 2, body, (z, z))
    a1, a2 = half(NCHUNK - 1, 0, carry)

    wait_wb(1)
    wait_wb(0)

    accv[0, :] = a1
    accv[1, :] = a2
    pltpu.sync_copy(accv, part_hbm.at[wid])

    plsc.subcore_barrier()
    pltpu.sync_copy(cnt_sh.at[pl.ds(r0, STRIPE)],
                    cnt_out.at[c_i, pl.ds(r0, STRIPE)])


def _k2(g, edge2, zcnt):
    mesh = plsc.VectorSubcoreMesh(core_axis_name="c", subcore_axis_name="s")
    f = pl.kernel(
        _k2_body,
        out_type=[
            jax.ShapeDtypeStruct((E, 16), jnp.float32),
            jax.ShapeDtypeStruct((32, 2, 16), jnp.float32),
            jax.ShapeDtypeStruct((2, NP), jnp.float32),
        ],
        mesh=mesh,
        scratch_types=[
            pltpu.VMEM((CHUNK // GB, GB), jnp.int32),
            pltpu.VMEM((CHUNK // GB, GB), jnp.int32),
            pltpu.VMEM((CHUNK // GB, GB), jnp.int32),
            pltpu.VMEM((CHUNK // GB, GB), jnp.int32),
            pltpu.VMEM((CHUNK, 16), jnp.float32),
            pltpu.VMEM((CHUNK, 16), jnp.float32),
            pltpu.VMEM((CHUNK, 16), jnp.float32),
            pltpu.VMEM((CHUNK, 16), jnp.float32),
            pltpu.VMEM((CHUNK, 16), jnp.float32),
            pltpu.VMEM((CHUNK, 16), jnp.float32),
            pltpu.VMEM((2, 16), jnp.float32),
            pltpu.VMEM((112,), jnp.float32),
            pltpu.VMEM_SHARED((NP,), jnp.float32),
            pltpu.SemaphoreType.DMA,
            pltpu.SemaphoreType.DMA,
            pltpu.SemaphoreType.DMA,
            pltpu.SemaphoreType.DMA,
            pltpu.SemaphoreType.DMA,
            pltpu.SemaphoreType.DMA,
            pltpu.SemaphoreType.DMA,
            pltpu.SemaphoreType.DMA,
        ],
        compiler_params=pltpu.CompilerParams(use_tc_tiling_on_sc=False),
    )
    return f(g, edge2, zcnt)


def _k3_body(y1_ref, p1_ref, w2_ref, bnp_ref, y2_ref, sq_ref, acc_ref):
    step = pl.program_id(0)

    @pl.when(step == 0)
    def _():
        acc_ref[...] = jnp.zeros_like(acc_ref)

    p1 = p1_ref[...]
    sums = jnp.sum(p1[:, :16], axis=0, keepdims=True)
    sqs = jnp.sum(p1[:, 16:], axis=0, keepdims=True)
    mu1 = sums / E
    var1 = sqs / E - mu1 * mu1
    g1 = bnp_ref[0:1, :]
    be1 = bnp_ref[1:2, :]
    s1 = g1 * lax.rsqrt(var1 + EPS)
    t1 = be1 - mu1 * s1

    y1 = y1_ref[...]
    segs = [jnp.maximum(y1[:, r * 16:(r + 1) * 16] * s1 + t1, 0.0)
            for r in range(8)]
    a1 = jnp.concatenate(segs, axis=1)
    z2 = jnp.dot(a1, w2_ref[...], preferred_element_type=jnp.float32)
    y2_ref[...] = z2
    acc_ref[0:1, :] += jnp.sum(z2, axis=0, keepdims=True)
    acc_ref[1:2, :] += jnp.sum(z2 * z2, axis=0, keepdims=True)

    s128 = acc_ref[0:1, :]
    q128 = acc_ref[1:2, :]
    sum16 = s128[:, 0:16]
    sq16 = q128[:, 0:16]
    for r in range(1, 8):
        sum16 = sum16 + s128[:, r * 16:(r + 1) * 16]
        sq16 = sq16 + q128[:, r * 16:(r + 1) * 16]
    mu2 = sum16 / E
    var2 = sq16 / E - mu2 * mu2
    g2 = bnp_ref[2:3, :]
    be2 = bnp_ref[3:4, :]
    s2 = g2 * lax.rsqrt(var2 + EPS)
    q2 = be2 - mu2 * s2
    sq_ref[...] = jnp.concatenate([s2, q2], axis=0)


def _k3(y1v, p1f, w2big, bnp):
    rows = E * 16 // 128
    blk = 8000
    nb = rows // blk
    return pl.pallas_call(
        _k3_body,
        grid=(nb,),
        in_specs=[
            pl.BlockSpec((blk, 128), lambda i: (i, 0)),
            pl.BlockSpec((32, 32), lambda i: (0, 0)),
            pl.BlockSpec((128, 128), lambda i: (0, 0)),
            pl.BlockSpec((4, 16), lambda i: (0, 0)),
        ],
        out_specs=[
            pl.BlockSpec((blk, 128), lambda i: (i, 0)),
            pl.BlockSpec((2, 16), lambda i: (0, 0)),
        ],
        out_shape=[
            jax.ShapeDtypeStruct((rows, 128), jnp.float32),
            jax.ShapeDtypeStruct((2, 16), jnp.float32),
        ],
        scratch_shapes=[pltpu.VMEM((8, 128), jnp.float32)],
    )(y1v, p1f, w2big, bnp)


def _k4_body(y2_hbm, edge2, s2q2, zrow, acc_out,
             acc_sh, ybuf0, ybuf1, idb0, idb1, sqv, semS0, semS1, semR):
    c_i = lax.axis_index("c")
    s_i = lax.axis_index("s")
    wid = s_i * 2 + c_i
    r0 = s_i * STRIPE
    RPC = CHUNK // GB

    ybuf = (ybuf0, ybuf1)
    idb = (idb0, idb1)
    semS = (semS0, semS1)

    pltpu.sync_copy(zrow.at[pl.ds(r0, STRIPE), :], acc_sh.at[pl.ds(r0, STRIPE), :])
    pltpu.sync_copy(s2q2, sqv)
    plsc.subcore_barrier()

    s2 = sqv[0, :]
    q2 = sqv[1, :]

    def wait_scatter(b):
        for j in range(RPC):
            pltpu.make_async_copy(y2_hbm.at[pl.ds(0, GB), :],
                                  ybuf[b].at[pl.ds(0, GB), :], semS[b]).wait()

    def half(c, b):
        @pl.when(c >= 2)
        def _():
            wait_scatter(b)

        eb = wid * EPW + c * CHUNK
        cps = [pltpu.async_copy(y2_hbm.at[pl.ds(eb, CHUNK), :], ybuf[b], semR)]
        for j in range(RPC):
            cps.append(pltpu.async_copy(
                edge2.at[1, pl.ds(eb + j * GB, GB)], idb[b].at[j], semR))
        for cp in cps:
            cp.wait()

        def inner(i, _2):
            for k in range(4):
                e = i * 4 + k
                ybuf[b][e, :] = jnp.maximum(ybuf[b][e, :] * s2 + q2, 0.0)
            return 0

        lax.fori_loop(0, CHUNK // 4, inner, 0)
        for j in range(RPC):
            pltpu.async_copy(ybuf[b].at[pl.ds(j * GB, GB), :],
                             acc_sh.at[idb[b].at[j]], semS[b], add=True)

    def body(c2, _):
        c = c2 * 2
        half(c, 0)
        half(c + 1, 1)
        return 0

    lax.fori_loop(0, (NCHUNK - 1) // 2, body, 0)
    half(NCHUNK - 1, 0)
    wait_scatter(1)
    wait_scatter(0)

    plsc.subcore_barrier()
    pltpu.sync_copy(acc_sh.at[pl.ds(r0, STRIPE), :],
                    acc_out.at[c_i, pl.ds(r0, STRIPE), :])


def _k4(y2, edge2, s2q2, zrow):
    mesh = plsc.VectorSubcoreMesh(core_axis_name="c", subcore_axis_name="s")
    f = pl.kernel(
        _k4_body,
        out_type=jax.ShapeDtypeStruct((2, NP, 16), jnp.float32),
        mesh=mesh,
        scratch_types=[
            pltpu.VMEM_SHARED((NP, 16), jnp.float32),
            pltpu.VMEM((CHUNK, 16), jnp.float32),
            pltpu.VMEM((CHUNK, 16), jnp.float32),
            pltpu.VMEM((CHUNK // GB, GB), jnp.int32),
            pltpu.VMEM((CHUNK // GB, GB), jnp.int32),
            pltpu.VMEM((2, 16), jnp.float32),
            pltpu.SemaphoreType.DMA,
            pltpu.SemaphoreType.DMA,
            pltpu.SemaphoreType.DMA,
        ],
        compiler_params=pltpu.CompilerParams(use_tc_tiling_on_sc=False),
    )
    return f(y2, edge2, s2q2, zrow)


_BLK5 = _NRP // 4


def _fold8(v2):
    out = v2[:, 0:16]
    for r in range(1, 8):
        out = out + v2[:, r * 16:(r + 1) * 16]
    return out


def _tile8(v):
    return jnp.concatenate([v] * 8, axis=1)


def _k5a_body(p_ref, v_ref, a0_ref, a1_ref, cb_ref, wap_ref, wav_ref, wb_ref,
              z_ref, st_ref, acc_ref):
    step = pl.program_id(0)

    @pl.when(step == 0)
    def _():
        acc_ref[...] = jnp.zeros_like(acc_ref)

    accs = a0_ref[...] + a1_ref[...]
    inv = 1.0 / jnp.maximum(cb_ref[...], 1.0)
    lane = lax.broadcasted_iota(jnp.int32, accs.shape, 1) % 16
    aggr = jnp.where(lane < 2, accs, accs * inv)
    z = (jnp.dot(p_ref[...], wap_ref[...], preferred_element_type=jnp.float32)
         + jnp.dot(v_ref[...], wav_ref[...], preferred_element_type=jnp.float32)
         + jnp.dot(aggr, wb_ref[...], preferred_element_type=jnp.float32))
    z_ref[...] = z
    acc_ref[0:1, :] += jnp.sum(z, axis=0, keepdims=True)
    acc_ref[1:2, :] += jnp.sum(z * z, axis=0, keepdims=True)
    st_ref[...] = acc_ref[...]


def _k5a(pos64p, vel64p, accall, cntb, wap, wav, wb):
    return pl.pallas_call(
        _k5a_body,
        grid=(4,),
        in_specs=[
            pl.BlockSpec((_BLK5, 64), lambda i: (i, 0)),
            pl.BlockSpec((_BLK5, 64), lambda i: (i, 0)),
            pl.BlockSpec((_BLK5, 128), lambda i: (i, 0)),
            pl.BlockSpec((_BLK5, 128), lambda i: (i + 4, 0)),
            pl.BlockSpec((_BLK5, 128), lambda i: (i, 0)),
            pl.BlockSpec((64, 128), lambda i: (0, 0)),
            pl.BlockSpec((64, 128), lambda i: (0, 0)),
            pl.BlockSpec((128, 128), lambda i: (0, 0)),
        ],
        out_specs=[
            pl.BlockSpec((_BLK5, 128), lambda i: (i, 0)),
            pl.BlockSpec((2, 128), lambda i: (0, 0)),
        ],
        out_shape=[
            jax.ShapeDtypeStruct((_NRP, 128), jnp.float32),
            jax.ShapeDtypeStruct((2, 128), jnp.float32),
        ],
        scratch_shapes=[pltpu.VMEM((2, 128), jnp.float32)],
    )(pos64p, vel64p, accall, accall, cntb, wap, wav, wb)


def _k5b_body(z_ref, st_ref, bn_ref, w2_ref, z2_ref, st2_ref, acc_ref):
    step = pl.program_id(0)

    @pl.when(step == 0)
    def _():
        acc_ref[...] = jnp.zeros_like(acc_ref)

    st = _fold8(st_ref[...])
    mu = st[0:1, :] / N
    var = st[1:2, :] / N - mu * mu
    s = bn_ref[0:1, :] * lax.rsqrt(var + EPS)
    t = bn_ref[1:2, :] - mu * s
    a = jnp.maximum(z_ref[...] * _tile8(s) + _tile8(t), 0.0)
    z2 = jnp.dot(a, w2_ref[...], preferred_element_type=jnp.float32)
    rowid = lax.broadcasted_iota(jnp.int32, z2.shape, 0) + step * _BLK5
    z2 = jnp.where(rowid < _NR, z2, 0.0)
    z2_ref[...] = z2
    acc_ref[0:1, :] += jnp.sum(z2, axis=0, keepdims=True)
    acc_ref[1:2, :] += jnp.sum(z2 * z2, axis=0, keepdims=True)
    st2_ref[...] = acc_ref[...]


def _k5b(z1, st1, ubn, w2u):
    return pl.pallas_call(
        _k5b_body,
        grid=(4,),
        in_specs=[
            pl.BlockSpec((_BLK5, 128), lambda i: (i, 0)),
            pl.BlockSpec((2, 128), lambda i: (0, 0)),
            pl.BlockSpec((4, 16), lambda i: (0, 0)),
            pl.BlockSpec((128, 128), lambda i: (0, 0)),
        ],
        out_specs=[
            pl.BlockSpec((_BLK5, 128), lambda i: (i, 0)),
            pl.BlockSpec((2, 128), lambda i: (0, 0)),
        ],
        out_shape=[
            jax.ShapeDtypeStruct((_NRP, 128), jnp.float32),
            jax.ShapeDtypeStruct((2, 128), jnp.float32),
        ],
        scratch_shapes=[pltpu.VMEM((2, 128), jnp.float32)],
    )(z1, st1, ubn, w2u)


def _k5c_body(z_ref, st_ref, bn_ref, pw_ref, pb_ref, out_ref):
    st = _fold8(st_ref[...])
    mu = st[0:1, :] / N
    var = st[1:2, :] / N - mu * mu
    s = bn_ref[2:3, :] * lax.rsqrt(var + EPS)
    t = bn_ref[3:4, :] - mu * s
    a = jnp.maximum(z_ref[...] * _tile8(s) + _tile8(t), 0.0)
    out_ref[...] = jnp.dot(a, pw_ref[...],
                           preferred_element_type=jnp.float32) + pb_ref[...]


def _k5c(z2, st2, ubn, pwb, pbt):
    return pl.pallas_call(
        _k5c_body,
        grid=(4,),
        in_specs=[
            pl.BlockSpec((_BLK5, 128), lambda i: (i, 0)),
            pl.BlockSpec((2, 128), lambda i: (0, 0)),
            pl.BlockSpec((4, 16), lambda i: (0, 0)),
            pl.BlockSpec((128, 16), lambda i: (0, 0)),
            pl.BlockSpec((1, 16), lambda i: (0, 0)),
        ],
        out_specs=pl.BlockSpec((_BLK5, 16), lambda i: (i, 0)),
        out_shape=jax.ShapeDtypeStruct((_NRP, 16), jnp.float32),
    )(z2, st2, ubn, pwb, pbt)


def kernel(pos, vel, edge_index, mW1, mb1, mg1, mbe1, mW2, mb2, mg2, mbe2,
           uW1, ub1, ug1, ube1, uW2, ub2, ug2, ube2, pW, pb):
    eye8 = jnp.eye(8, dtype=jnp.float32)
    pos64 = pos.reshape(_NR, 64)
    vel64 = vel.reshape(_NR, 64)
    g128 = _k1(pos64, vel64,
               jnp.kron(eye8, mW1[:8, :]), jnp.kron(eye8, mW1[8:, :]))
    g = g128.reshape(N, 16)

    zcnt = jnp.zeros((NP,), jnp.float32)
    y1, part1, cntp = _k2(g, edge_index, zcnt)

    y1v = y1.reshape(E * 16 // 128, 128)
    p1f = part1.reshape(32, 32)
    w2big = jnp.kron(eye8, mW2)
    bnp = jnp.stack([mg1, mbe1, mg2, mbe2])
    y2v, s2q2 = _k3(y1v, p1f, w2big, bnp)
    y2 = y2v.reshape(E, 16)

    zrow = jnp.zeros((NP, 16), jnp.float32)
    accp = _k4(y2, edge_index, s2q2, zrow)

    ubn = jnp.stack([ug1, ube1, ug2, ube2])
    zpad = jnp.zeros((_NRP - _NR, 64), jnp.float32)
    pos64p = jnp.concatenate([pos64, zpad], axis=0)
    vel64p = jnp.concatenate([vel64, zpad], axis=0)
    accall = accp.reshape(2 * _NRP, 128)
    cnt_t = cntp[0] + cntp[1]
    cntb = jnp.repeat(cnt_t, 16).reshape(_NRP, 128)
    wap = jnp.kron(eye8, uW1[:8, :])
    wav = jnp.kron(eye8, uW1[8:16, :])
    wb = jnp.kron(eye8, uW1[16:, :])
    z1, st1 = _k5a(pos64p, vel64p, accall, cntb, wap, wav, wb)
    z2, st2 = _k5b(z1, st1, ubn, jnp.kron(eye8, uW2))
    out128 = _k5c(z2, st2, ubn, jnp.kron(eye8, pW), jnp.tile(pb, 8).reshape(1, 16))
    return out128[:_NR].reshape(N, 2)
